# baseline (speedup 1.0000x reference)
import functools

import jax
import jax.numpy as jnp
from jax import lax
from jax.experimental import pallas as pl
from jax.experimental.pallas import tpu as pltpu
from jax.experimental.pallas import tpu_sc as plsc

B = 16384
D = 64
H = 128
V = 1000000
QR = 524288
Q2 = 262144
WB = 4096
P0 = 131072
P1 = 155648
S = P1 - P0
NC = 2
NS = 16
NW = NC * NS
BPW = B // NW
CH = BPW // 2
RPT = 2 * S // NW
NCH = RPT // 128
KSKIP = P1 // WB - P0 // WB


def _tr_body(a_ref, b_ref, out_ref):
    a16 = a_ref[...].T.astype(jnp.bfloat16)
    b16 = b_ref[...].T.astype(jnp.bfloat16)
    pa = pltpu.bitcast(a16, jnp.float32)
    pb = pltpu.bitcast(b16, jnp.float32)
    out_ref[...] = jnp.concatenate([pa, pb], axis=1)


def _pair_table(tabT):
    nlast = V // (2 * WB)
    k0 = P0 // WB

    def remap(i):
        return jnp.where(i < k0, i, i + KSKIP)

    return pl.pallas_call(
        _tr_body,
        grid=(Q2 // WB - KSKIP,),
        in_specs=[
            pl.BlockSpec((D, 2 * WB), lambda i: (0, remap(i))),
            pl.BlockSpec((D, 2 * WB),
                         lambda i: (0, jnp.minimum(remap(i) + QR // (2 * WB), nlast))),
        ],
        out_specs=pl.BlockSpec((WB, 128), lambda i: (remap(i), 0)),
        out_shape=jax.ShapeDtypeStruct((Q2, 128), jnp.float32),
    )(tabT, tabT)


def _sc_tr_body(tabT, out_hbm, in0, in1, ot0, ot1, sin0, sin1, sot0, sot1):
    wid = lax.axis_index("s") * NC + lax.axis_index("c")
    reg = wid // 16
    lt = wid % 16
    colbase = 2 * P0 + reg * QR + lt * (2 * RPT)
    rowbase = reg * S + lt * RPT

    ins = (in0, in1)
    outs = (ot0, ot1)
    sins = (sin0, sin1)
    souts = (sot0, sot1)

    rowpat = lax.iota(jnp.int32, 16) >> 1
    colpat = (lax.iota(jnp.int32, 16) & 1) * 64

    def compute(inbuf, outbuf):
        def feat(f, _):
            cols = colpat + f
            for c in range(16):
                v = inbuf[f, pl.ds(16 * c, 16)]
                plsc.store_scatter(outbuf, [8 * c + rowpat, cols], v)
            return 0
        lax.fori_loop(0, D, feat, 0)

    def chunk(c, b, nxt):
        @pl.when(c + 1 < NCH)
        def _():
            pltpu.async_copy(
                tabT.at[:, pl.ds(colbase + (c + 1) * 256, 256)], ins[nxt], sins[nxt])
        pltpu.make_async_copy(
            tabT.at[:, pl.ds(colbase, 256)], ins[b], sins[b]).wait()
        compute(ins[b], outs[b])
        pltpu.async_copy(outs[b].at[:, pl.ds(0, 128)],
                         out_hbm.at[pl.ds(rowbase + c * 128, 128)],
                         souts[b]).wait()

    pltpu.async_copy(tabT.at[:, pl.ds(colbase, 256)], ins[0], sins[0])

    def two(cc, _):
        chunk(cc * 2, 0, 1)
        chunk(cc * 2 + 1, 1, 0)
        return 0
    lax.fori_loop(0, NCH // 2, two, 0)


@functools.cache
def _sc_transpose():
    return pl.kernel(
        _sc_tr_body,
        out_type=jax.ShapeDtypeStruct((2 * S, 128), jnp.float32),
        mesh=plsc.VectorSubcoreMesh(core_axis_name="c", subcore_axis_name="s"),
        scratch_types=[
            pltpu.VMEM((D, 256), jnp.float32),
            pltpu.VMEM((D, 256), jnp.float32),
            pltpu.VMEM((128, 129), jnp.float32),
            pltpu.VMEM((128, 129), jnp.float32),
            pltpu.SemaphoreType.DMA,
            pltpu.SemaphoreType.DMA,
            pltpu.SemaphoreType.DMA,
            pltpu.SemaphoreType.DMA,
        ],
        compiler_params=pltpu.CompilerParams(needs_layout_passes=False),
    )


def _gather_steps(idx_hbm, tables, e_out, idxs, bufs, sems, base):
    for t in range(len(tables)):
        pltpu.sync_copy(idx_hbm.at[pl.ds(t * B + base, BPW)], idxs[t])
    pending = [None, None]
    dst = [None, None]
    step = 0
    for t in range(len(tables)):
        for c in range(2):
            s = step % 2
            if pending[s] is not None:
                pending[s].wait()
                pltpu.sync_copy(bufs[s], e_out.at[dst[s][0], pl.ds(dst[s][1], CH)])
            pending[s] = pltpu.async_copy(
                tables[t].at[idxs[t].at[pl.ds(c * CH, CH)]], bufs[s], sems[s])
            dst[s] = (t, base + c * CH)
            step += 1
    for s in range(2):
        pending[s].wait()
        pltpu.sync_copy(bufs[s], e_out.at[dst[s][0], pl.ds(dst[s][1], CH)])


def _sc_gather_cats_body(cat_idx, c1_t, c2_t, c3_t, e_out,
                         idx0, idx1, idx2, rows0, rows1, sem0, sem1):
    wid = lax.axis_index("s") * NC + lax.axis_index("c")
    _gather_steps(cat_idx, (c1_t, c2_t, c3_t), e_out,
                  (idx0, idx1, idx2), (rows0, rows1), (sem0, sem1), wid * BPW)


def _sc_gather_item_body(item_idx, quad_t, scp_t, e_out,
                         idx0, idx1, rows0, rows1, sem0, sem1):
    wid = lax.axis_index("s") * NC + lax.axis_index("c")
    _gather_steps(item_idx, (quad_t, scp_t), e_out,
                  (idx0, idx1), (rows0, rows1), (sem0, sem1), wid * BPW)


@functools.cache
def _sc_gather_cats():
    return pl.kernel(
        _sc_gather_cats_body,
        out_type=jax.ShapeDtypeStruct((3, B, 2 * D), jnp.float32),
        mesh=plsc.VectorSubcoreMesh(core_axis_name="c", subcore_axis_name="s"),
        scratch_types=[
            pltpu.VMEM((BPW,), jnp.int32),
            pltpu.VMEM((BPW,), jnp.int32),
            pltpu.VMEM((BPW,), jnp.int32),
            pltpu.VMEM((CH, 2 * D), jnp.float32),
            pltpu.VMEM((CH, 2 * D), jnp.float32),
            pltpu.SemaphoreType.DMA,
            pltpu.SemaphoreType.DMA,
        ],
    )


@functools.cache
def _sc_gather_item():
    return pl.kernel(
        _sc_gather_item_body,
        out_type=jax.ShapeDtypeStruct((2, B, 2 * D), jnp.float32),
        mesh=plsc.VectorSubcoreMesh(core_axis_name="c", subcore_axis_name="s"),
        scratch_types=[
            pltpu.VMEM((BPW,), jnp.int32),
            pltpu.VMEM((BPW,), jnp.int32),
            pltpu.VMEM((CH, 2 * D), jnp.float32),
            pltpu.VMEM((CH, 2 * D), jnp.float32),
            pltpu.SemaphoreType.DMA,
            pltpu.SemaphoreType.DMA,
        ],
    )


def _half(x, bit):
    return jnp.where(bit[:, None] == 1, x[:, D:2 * D], x[:, 0:D])


def _mlp_body(ei_ref, ec_ref, par_ref, w1_ref, b1_ref, w2_ref, b2_ref, out_ref):
    xh0 = _half(ei_ref[0], par_ref[0])
    u = jax.lax.bitcast_convert_type(xh0, jnp.int32)
    chosen = jnp.where(par_ref[1][:, None] == 1, u & jnp.int32(-65536), u << 16)
    x0 = jax.lax.bitcast_convert_type(chosen, jnp.float32)
    x1 = _half(ei_ref[1], par_ref[1])
    xi = jnp.where(par_ref[2][:, None] == 1, x1, x0)

    h = jnp.dot(xi, w1_ref[0:D], preferred_element_type=jnp.float32)
    for t in range(3):
        ct = _half(ec_ref[t], par_ref[t + 3])
        h += jnp.dot(ct, w1_ref[(t + 1) * D:(t + 2) * D],
                     preferred_element_type=jnp.float32)
    h = jnp.maximum(h + b1_ref[...], 0.0)
    out_ref[...] = jnp.dot(h, w2_ref[...], preferred_element_type=jnp.float32) + b2_ref[...]


def _mlp(ei, ec, par, w1, b1, w2, b2, blk=2048):
    return pl.pallas_call(
        _mlp_body,
        grid=(B // blk,),
        in_specs=[
            pl.BlockSpec((2, blk, 2 * D), lambda i: (0, i, 0)),
            pl.BlockSpec((3, blk, 2 * D), lambda i: (0, i, 0)),
            pl.BlockSpec((6, blk), lambda i: (0, i)),
            pl.BlockSpec((4 * D, H), lambda i: (0, 0)),
            pl.BlockSpec((1, H), lambda i: (0, 0)),
            pl.BlockSpec((H, D), lambda i: (0, 0)),
            pl.BlockSpec((1, D), lambda i: (0, 0)),
        ],
        out_specs=pl.BlockSpec((blk, D), lambda i: (i, 0)),
        out_shape=jax.ShapeDtypeStruct((B, D), jnp.float32),
    )(ei, ec, par, w1, b1, w2, b2)


def kernel(item_id, category, category2, category3,
           item_table, cat1_table, cat2_table, cat3_table,
           W1, b1, W2, b2):
    tabT = item_table.T
    sc_pairs = _sc_transpose()(tabT)

    cat_idx = jnp.stack([category >> 1, category2 >> 1, category3 >> 1]).reshape(-1)
    ec = _sc_gather_cats()(cat_idx,
                           cat1_table.reshape(-1, 2 * D),
                           cat2_table.reshape(-1, 2 * D),
                           cat3_table.reshape(-1, 2 * D))

    quad = _pair_table(tabT)

    reg = (item_id >= QR).astype(jnp.int32)
    loc = item_id - QR * reg
    p = loc >> 1
    hb = ((p >= P0) & (p < P1)).astype(jnp.int32)
    sc_row = jnp.where(hb == 1, p - P0 + S * reg, p % S)
    item_idx = jnp.stack([p, sc_row]).reshape(-1)

    ei = _sc_gather_item()(item_idx, quad, sc_pairs)

    par = jnp.stack([reg, loc & 1, hb,
                     category & 1, category2 & 1, category3 & 1])
    return _mlp(ei, ec, par, W1, b1.reshape(1, H), W2, b2.reshape(1, D))


# speedup vs baseline: 1.0910x; 1.0910x over previous
import functools

import jax
import jax.numpy as jnp
from jax import lax
from jax.experimental import pallas as pl
from jax.experimental.pallas import tpu as pltpu
from jax.experimental.pallas import tpu_sc as plsc

B = 16384
D = 64
H = 128
V = 1000000
QR = 524288
Q2 = 262144
WB = 8192
NC = 2
NS = 16
NW = NC * NS
BPW = B // NW
CH = BPW // 2


def _tr_body(a_ref, b_ref, out_ref):
    a16 = a_ref[...].T.astype(jnp.bfloat16)
    b16 = b_ref[...].T.astype(jnp.bfloat16)
    pa = pltpu.bitcast(a16, jnp.float32)
    pb = pltpu.bitcast(b16, jnp.float32)
    out_ref[...] = jnp.concatenate([pa, pb], axis=1)


def _pair_table(tabT):
    nlast = V // (2 * WB)
    return pl.pallas_call(
        _tr_body,
        grid=(Q2 // WB,),
        in_specs=[
            pl.BlockSpec((D, 2 * WB), lambda i: (0, i)),
            pl.BlockSpec((D, 2 * WB),
                         lambda i: (0, jnp.minimum(i + QR // (2 * WB), nlast))),
        ],
        out_specs=pl.BlockSpec((WB, 128), lambda i: (i, 0)),
        out_shape=jax.ShapeDtypeStruct((Q2, 128), jnp.float32),
    )(tabT, tabT)


def _gather_steps(idx_hbm, tables, e_out, idxs, bufs, sems, base):
    for t in range(len(tables)):
        pltpu.sync_copy(idx_hbm.at[pl.ds(t * B + base, BPW)], idxs[t])
    pending = [None, None]
    dst = [None, None]
    step = 0
    for t in range(len(tables)):
        for c in range(2):
            s = step % 2
            if pending[s] is not None:
                pending[s].wait()
                pltpu.sync_copy(bufs[s], e_out.at[dst[s][0], pl.ds(dst[s][1], CH)])
            pending[s] = pltpu.async_copy(
                tables[t].at[idxs[t].at[pl.ds(c * CH, CH)]], bufs[s], sems[s])
            dst[s] = (t, base + c * CH)
            step += 1
    for s in range(2):
        pending[s].wait()
        pltpu.sync_copy(bufs[s], e_out.at[dst[s][0], pl.ds(dst[s][1], CH)])


def _sc_gather_cats_body(cat_idx, c1_t, c2_t, c3_t, e_out,
                         idx0, idx1, idx2, rows0, rows1, sem0, sem1):
    wid = lax.axis_index("s") * NC + lax.axis_index("c")
    _gather_steps(cat_idx, (c1_t, c2_t, c3_t), e_out,
                  (idx0, idx1, idx2), (rows0, rows1), (sem0, sem1), wid * BPW)


def _sc_gather_item_body(item_idx, quad_t, e_out,
                         idx0, rows0, rows1, sem0, sem1):
    wid = lax.axis_index("s") * NC + lax.axis_index("c")
    base = wid * BPW
    pltpu.sync_copy(item_idx.at[pl.ds(base, BPW)], idx0)
    cp0 = pltpu.async_copy(quad_t.at[idx0.at[pl.ds(0, CH)]], rows0, sem0)
    cp1 = pltpu.async_copy(quad_t.at[idx0.at[pl.ds(CH, CH)]], rows1, sem1)
    cp0.wait()
    pltpu.sync_copy(rows0, e_out.at[pl.ds(base, CH)])
    cp1.wait()
    pltpu.sync_copy(rows1, e_out.at[pl.ds(base + CH, CH)])


@functools.cache
def _sc_gather_cats():
    return pl.kernel(
        _sc_gather_cats_body,
        out_type=jax.ShapeDtypeStruct((3, B, 2 * D), jnp.float32),
        mesh=plsc.VectorSubcoreMesh(core_axis_name="c", subcore_axis_name="s"),
        scratch_types=[
            pltpu.VMEM((BPW,), jnp.int32),
            pltpu.VMEM((BPW,), jnp.int32),
            pltpu.VMEM((BPW,), jnp.int32),
            pltpu.VMEM((CH, 2 * D), jnp.float32),
            pltpu.VMEM((CH, 2 * D), jnp.float32),
            pltpu.SemaphoreType.DMA,
            pltpu.SemaphoreType.DMA,
        ],
    )


@functools.cache
def _sc_gather_item():
    return pl.kernel(
        _sc_gather_item_body,
        out_type=jax.ShapeDtypeStruct((B, 2 * D), jnp.float32),
        mesh=plsc.VectorSubcoreMesh(core_axis_name="c", subcore_axis_name="s"),
        scratch_types=[
            pltpu.VMEM((BPW,), jnp.int32),
            pltpu.VMEM((CH, 2 * D), jnp.float32),
            pltpu.VMEM((CH, 2 * D), jnp.float32),
            pltpu.SemaphoreType.DMA,
            pltpu.SemaphoreType.DMA,
        ],
    )


def _half(x, bit):
    return jnp.where(bit[:, None] == 1, x[:, D:2 * D], x[:, 0:D])


def _mlp_body(ei_ref, ec_ref, par_ref, w1_ref, b1_ref, w2_ref, b2_ref, out_ref):
    xh0 = _half(ei_ref[...], par_ref[0])
    u = jax.lax.bitcast_convert_type(xh0, jnp.int32)
    chosen = jnp.where(par_ref[1][:, None] == 1, u & jnp.int32(-65536), u << 16)
    xi = jax.lax.bitcast_convert_type(chosen, jnp.float32)

    h = jnp.dot(xi, w1_ref[0:D], preferred_element_type=jnp.float32)
    for t in range(3):
        ct = _half(ec_ref[t], par_ref[t + 2])
        h += jnp.dot(ct, w1_ref[(t + 1) * D:(t + 2) * D],
                     preferred_element_type=jnp.float32)
    h = jnp.maximum(h + b1_ref[...], 0.0)
    out_ref[...] = jnp.dot(h, w2_ref[...], preferred_element_type=jnp.float32) + b2_ref[...]


def _mlp(ei, ec, par, w1, b1, w2, b2, blk=2048):
    return pl.pallas_call(
        _mlp_body,
        grid=(B // blk,),
        in_specs=[
            pl.BlockSpec((blk, 2 * D), lambda i: (i, 0)),
            pl.BlockSpec((3, blk, 2 * D), lambda i: (0, i, 0)),
            pl.BlockSpec((5, blk), lambda i: (0, i)),
            pl.BlockSpec((4 * D, H), lambda i: (0, 0)),
            pl.BlockSpec((1, H), lambda i: (0, 0)),
            pl.BlockSpec((H, D), lambda i: (0, 0)),
            pl.BlockSpec((1, D), lambda i: (0, 0)),
        ],
        out_specs=pl.BlockSpec((blk, D), lambda i: (i, 0)),
        out_shape=jax.ShapeDtypeStruct((B, D), jnp.float32),
    )(ei, ec, par, w1, b1, w2, b2)


def kernel(item_id, category, category2, category3,
           item_table, cat1_table, cat2_table, cat3_table,
           W1, b1, W2, b2):
    cat_idx = jnp.stack([category >> 1, category2 >> 1, category3 >> 1]).reshape(-1)
    ec = _sc_gather_cats()(cat_idx,
                           cat1_table.reshape(-1, 2 * D),
                           cat2_table.reshape(-1, 2 * D),
                           cat3_table.reshape(-1, 2 * D))

    quad = _pair_table(item_table.T)

    reg = (item_id >= QR).astype(jnp.int32)
    loc = item_id - QR * reg
    ei = _sc_gather_item()(loc >> 1, quad)

    par = jnp.stack([reg, loc & 1, category & 1, category2 & 1, category3 & 1])
    return _mlp(ei, ec, par, W1, b1.reshape(1, H), W2, b2.reshape(1, D))


# speedup vs baseline: 1.1028x; 1.0108x over previous
import functools

import jax
import jax.numpy as jnp
from jax import lax
from jax.experimental import pallas as pl
from jax.experimental.pallas import tpu as pltpu
from jax.experimental.pallas import tpu_sc as plsc

B = 16384
D = 64
H = 128
V = 1000000
QR = 524288
Q2 = 262144
WB = 16384
NC = 2
NS = 16
NW = NC * NS
BPW = B // NW
CH = BPW // 2


def _tr_body(a_ref, b_ref, out_ref):
    a16 = a_ref[...].T.astype(jnp.bfloat16)
    b16 = b_ref[...].T.astype(jnp.bfloat16)
    pa = pltpu.bitcast(a16, jnp.float32)
    pb = pltpu.bitcast(b16, jnp.float32)
    out_ref[...] = jnp.concatenate([pa, pb], axis=1)


def _pair_table(tabT):
    nlast = V // (2 * WB)
    return pl.pallas_call(
        _tr_body,
        grid=(Q2 // WB,),
        in_specs=[
            pl.BlockSpec((D, 2 * WB), lambda i: (0, i)),
            pl.BlockSpec((D, 2 * WB),
                         lambda i: (0, jnp.minimum(i + QR // (2 * WB), nlast))),
        ],
        out_specs=pl.BlockSpec((WB, 128), lambda i: (i, 0)),
        out_shape=jax.ShapeDtypeStruct((Q2, 128), jnp.float32),
        compiler_params=pltpu.CompilerParams(vmem_limit_bytes=112 * 1024 * 1024),
    )(tabT, tabT)


def _gather_steps(idx_hbm, tables, e_out, idxs, bufs, sems, base):
    for t in range(len(tables)):
        pltpu.sync_copy(idx_hbm.at[pl.ds(t * B + base, BPW)], idxs[t])
    pending = [None, None]
    dst = [None, None]
    step = 0
    for t in range(len(tables)):
        for c in range(2):
            s = step % 2
            if pending[s] is not None:
                pending[s].wait()
                pltpu.sync_copy(bufs[s], e_out.at[dst[s][0], pl.ds(dst[s][1], CH)])
            pending[s] = pltpu.async_copy(
                tables[t].at[idxs[t].at[pl.ds(c * CH, CH)]], bufs[s], sems[s])
            dst[s] = (t, base + c * CH)
            step += 1
    for s in range(2):
        pending[s].wait()
        pltpu.sync_copy(bufs[s], e_out.at[dst[s][0], pl.ds(dst[s][1], CH)])


def _sc_gather_cats_body(cat_idx, c1_t, c2_t, c3_t, e_out,
                         idx0, idx1, idx2, rows0, rows1, sem0, sem1):
    wid = lax.axis_index("s") * NC + lax.axis_index("c")
    _gather_steps(cat_idx, (c1_t, c2_t, c3_t), e_out,
                  (idx0, idx1, idx2), (rows0, rows1), (sem0, sem1), wid * BPW)


def _sc_gather_item_body(item_idx, quad_t, e_out,
                         idx0, rows0, rows1, sem0, sem1):
    wid = lax.axis_index("s") * NC + lax.axis_index("c")
    base = wid * BPW
    pltpu.sync_copy(item_idx.at[pl.ds(base, BPW)], idx0)
    cp0 = pltpu.async_copy(quad_t.at[idx0.at[pl.ds(0, CH)]], rows0, sem0)
    cp1 = pltpu.async_copy(quad_t.at[idx0.at[pl.ds(CH, CH)]], rows1, sem1)
    cp0.wait()
    pltpu.sync_copy(rows0, e_out.at[pl.ds(base, CH)])
    cp1.wait()
    pltpu.sync_copy(rows1, e_out.at[pl.ds(base + CH, CH)])


@functools.cache
def _sc_gather_cats():
    return pl.kernel(
        _sc_gather_cats_body,
        out_type=jax.ShapeDtypeStruct((3, B, 2 * D), jnp.float32),
        mesh=plsc.VectorSubcoreMesh(core_axis_name="c", subcore_axis_name="s"),
        scratch_types=[
            pltpu.VMEM((BPW,), jnp.int32),
            pltpu.VMEM((BPW,), jnp.int32),
            pltpu.VMEM((BPW,), jnp.int32),
            pltpu.VMEM((CH, 2 * D), jnp.float32),
            pltpu.VMEM((CH, 2 * D), jnp.float32),
            pltpu.SemaphoreType.DMA,
            pltpu.SemaphoreType.DMA,
        ],
    )


@functools.cache
def _sc_gather_item():
    return pl.kernel(
        _sc_gather_item_body,
        out_type=jax.ShapeDtypeStruct((B, 2 * D), jnp.float32),
        mesh=plsc.VectorSubcoreMesh(core_axis_name="c", subcore_axis_name="s"),
        scratch_types=[
            pltpu.VMEM((BPW,), jnp.int32),
            pltpu.VMEM((CH, 2 * D), jnp.float32),
            pltpu.VMEM((CH, 2 * D), jnp.float32),
            pltpu.SemaphoreType.DMA,
            pltpu.SemaphoreType.DMA,
        ],
    )


def _half(x, bit):
    return jnp.where(bit[:, None] == 1, x[:, D:2 * D], x[:, 0:D])


def _mlp_body(ei_ref, ec_ref, par_ref, w1_ref, b1_ref, w2_ref, b2_ref, out_ref):
    xh0 = _half(ei_ref[...], par_ref[0])
    u = jax.lax.bitcast_convert_type(xh0, jnp.int32)
    chosen = jnp.where(par_ref[1][:, None] == 1, u & jnp.int32(-65536), u << 16)
    xi = jax.lax.bitcast_convert_type(chosen, jnp.float32)

    h = jnp.dot(xi, w1_ref[0:D], preferred_element_type=jnp.float32)
    for t in range(3):
        ct = _half(ec_ref[t], par_ref[t + 2])
        h += jnp.dot(ct, w1_ref[(t + 1) * D:(t + 2) * D],
                     preferred_element_type=jnp.float32)
    h = jnp.maximum(h + b1_ref[...], 0.0)
    out_ref[...] = jnp.dot(h, w2_ref[...], preferred_element_type=jnp.float32) + b2_ref[...]


def _mlp(ei, ec, par, w1, b1, w2, b2, blk=2048):
    return pl.pallas_call(
        _mlp_body,
        grid=(B // blk,),
        in_specs=[
            pl.BlockSpec((blk, 2 * D), lambda i: (i, 0)),
            pl.BlockSpec((3, blk, 2 * D), lambda i: (0, i, 0)),
            pl.BlockSpec((5, blk), lambda i: (0, i)),
            pl.BlockSpec((4 * D, H), lambda i: (0, 0)),
            pl.BlockSpec((1, H), lambda i: (0, 0)),
            pl.BlockSpec((H, D), lambda i: (0, 0)),
            pl.BlockSpec((1, D), lambda i: (0, 0)),
        ],
        out_specs=pl.BlockSpec((blk, D), lambda i: (i, 0)),
        out_shape=jax.ShapeDtypeStruct((B, D), jnp.float32),
    )(ei, ec, par, w1, b1, w2, b2)


def kernel(item_id, category, category2, category3,
           item_table, cat1_table, cat2_table, cat3_table,
           W1, b1, W2, b2):
    cat_idx = jnp.stack([category >> 1, category2 >> 1, category3 >> 1]).reshape(-1)
    ec = _sc_gather_cats()(cat_idx,
                           cat1_table.reshape(-1, 2 * D),
                           cat2_table.reshape(-1, 2 * D),
                           cat3_table.reshape(-1, 2 * D))

    quad = _pair_table(item_table.T)

    reg = (item_id >= QR).astype(jnp.int32)
    loc = item_id - QR * reg
    ei = _sc_gather_item()(loc >> 1, quad)

    par = jnp.stack([reg, loc & 1, category & 1, category2 & 1, category3 & 1])
    return _mlp(ei, ec, par, W1, b1.reshape(1, H), W2, b2.reshape(1, D))
